# baseline (speedup 1.0000x reference)
import functools

import jax
import jax.numpy as jnp
from jax import lax
from jax.experimental import pallas as pl
from jax.experimental.pallas import tpu as pltpu
from jax.experimental.pallas import tpu_sc as plsc

B = 4096
F = 26
V = 100000
D = 64
H1 = 256
H2 = 128
EPS = 1e-5

NC = 2
NS = 16
NW = NC * NS
CHUNK = B // NW
S = 32
NSUB = CHUNK // S


def _sc_fm_body(idx_hbm, emb_hbm, lin_hbm, fm_out, lin_out,
                idx_v, buf_v, linbuf_v, fm_stage, lin_stage, sem, lsem):
    wid = lax.axis_index("s") * NC + lax.axis_index("c")
    base = wid * CHUNK

    pltpu.sync_copy(idx_hbm.at[wid], idx_v)

    lin_handles = []
    for f in range(F):
        lin_handles.append(
            pltpu.async_copy(lin_hbm.at[idx_v.at[f]], linbuf_v.at[f], lsem))

    for sub in range(NSUB):
        r0 = sub * S
        handles = []
        for f in range(F):
            handles.append(
                pltpu.async_copy(
                    emb_hbm.at[idx_v.at[f, pl.ds(r0, S)]], buf_v.at[f], sem))
        for h in handles:
            h.wait()

        def row_body(i, _):
            s = []
            q = []
            for d in range(D // 16):
                r = buf_v[0, i, pl.ds(d * 16, 16)]
                s.append(r)
                q.append(r * r)
            for f in range(1, F):
                for d in range(D // 16):
                    r = buf_v[f, i, pl.ds(d * 16, 16)]
                    s[d] = s[d] + r
                    q[d] = q[d] + r * r
            for d in range(D // 16):
                fm_stage[r0 + i, pl.ds(d * 16, 16)] = 0.5 * (s[d] * s[d] - q[d])
            return 0

        lax.fori_loop(0, S, row_body, 0)

    for h in lin_handles:
        h.wait()
    for rc in range(CHUNK // 16):
        acc = linbuf_v[0, pl.ds(rc * 16, 16)]
        for f in range(1, F):
            acc = acc + linbuf_v[f, pl.ds(rc * 16, 16)]
        lin_stage[pl.ds(rc * 16, 16)] = acc

    pltpu.sync_copy(fm_stage, fm_out.at[pl.ds(base, CHUNK)])
    pltpu.sync_copy(lin_stage, lin_out.at[pl.ds(base, CHUNK)])


def _sc_fm(idx, emb_flat, lin_flat):
    mesh = plsc.VectorSubcoreMesh(
        core_axis_name="c", subcore_axis_name="s", num_cores=NC,
        num_subcores=NS)
    return pl.kernel(
        _sc_fm_body,
        out_type=[
            jax.ShapeDtypeStruct((B, D), jnp.float32),
            jax.ShapeDtypeStruct((B,), jnp.float32),
        ],
        mesh=mesh,
        compiler_params=pltpu.CompilerParams(use_tc_tiling_on_sc=False),
        scratch_types=[
            pltpu.VMEM((F, CHUNK), jnp.int32),
            pltpu.VMEM((F, S, D), jnp.float32),
            pltpu.VMEM((F, CHUNK), jnp.float32),
            pltpu.VMEM((CHUNK, D), jnp.float32),
            pltpu.VMEM((CHUNK,), jnp.float32),
            pltpu.SemaphoreType.DMA,
            pltpu.SemaphoreType.DMA,
        ],
    )(idx, emb_flat, lin_flat)


def _mlp_body(fm, lin, w1t, a1, d1, w2t, a2, d2, wv, out):
    h = jnp.dot(fm[...], w1t[...], preferred_element_type=jnp.float32)
    h = jnp.maximum(h * a1[...] + d1[...], 0.0)
    h = jnp.dot(h, w2t[...], preferred_element_type=jnp.float32)
    h = jnp.maximum(h * a2[...] + d2[...], 0.0)
    o = jnp.dot(h, wv[...], preferred_element_type=jnp.float32)
    z = lin[...] + o
    out[...] = 1.0 / (1.0 + jnp.exp(-z))


def _mlp(fm, lin2, w1t, a1, d1, w2t, a2, d2, wv):
    return pl.pallas_call(
        _mlp_body,
        out_shape=jax.ShapeDtypeStruct((B, 1), jnp.float32),
    )(fm, lin2, w1t, a1, d1, w2t, a2, d2, wv)


@jax.jit
def kernel(x, emb_tables, lin_tables, bias, W1, b1, g1, be1, rm1, rv1,
           W2, b2, g2, be2, rm2, rv2, Wout, bout):
    offs = jnp.arange(F, dtype=jnp.int32) * V
    idx = (x + offs[None, :]).T.reshape(F, NW, CHUNK).transpose(1, 0, 2)
    emb_flat = emb_tables.reshape(F * V, D)
    lin_flat = lin_tables.reshape(F * V)

    fm, lin_sum = _sc_fm(idx, emb_flat, lin_flat)

    a1 = (g1 / jnp.sqrt(rv1 + EPS)).reshape(1, H1)
    d1 = ((b1 - rm1) * a1[0] + be1).reshape(1, H1)
    a2 = (g2 / jnp.sqrt(rv2 + EPS)).reshape(1, H2)
    d2 = ((b2 - rm2) * a2[0] + be2).reshape(1, H2)
    lin2 = (lin_sum + bias[0] + bout[0]).reshape(B, 1)

    return _mlp(fm, lin2, W1.T, a1, d1, W2.T, a2, d2, Wout.T)
